# baseline (speedup 1.0000x reference)
import functools

import jax
import jax.numpy as jnp
from jax import lax
from jax.experimental import pallas as pl
from jax.experimental.pallas import tpu as pltpu
from jax.experimental.pallas import tpu_sc as plsc

_KT = 512
_RB = 256


def _argmin_body(kt, rb, z_ref, e_ref, idx_ref, min_sc, arg_sc):
    k = pl.program_id(0)
    s = pl.program_id(1)

    zs = z_ref[...]
    e_full = e_ref[...]
    d = zs.shape[1]
    e_sq = e_full * e_full
    e_half = 0.5 * lax.dot_general(
        jnp.ones((8, d), jnp.float32), e_sq, (((1,), (1,)), ((), ())),
        preferred_element_type=jnp.float32,
        precision=lax.Precision.HIGHEST,
    )[0:1, :]
    cross = lax.dot_general(
        zs.astype(jnp.bfloat16), e_full.astype(jnp.bfloat16),
        (((1,), (1,)), ((), ())),
        preferred_element_type=jnp.float32,
    )

    rm = e_half[:, 0:128] - cross[:, 0:128]
    ri = lax.broadcasted_iota(jnp.int32, (rb, 128), 1) + k * kt
    for c in range(1, kt // 128):
        col0 = c * 128
        sc = e_half[:, col0:col0 + 128] - cross[:, col0:col0 + 128]
        col = lax.broadcasted_iota(jnp.int32, (rb, 128), 1) + (k * kt + col0)
        m = sc < rm
        rm = jnp.where(m, sc, rm)
        ri = jnp.where(m, col, ri)
    block_min = jnp.min(rm, axis=1, keepdims=True)
    block_arg = jnp.min(
        jnp.where(rm == block_min, ri, jnp.int32(2**30)),
        axis=1,
        keepdims=True,
    )

    row = pl.ds(s * rb, rb)

    @pl.when(k == 0)
    def _():
        min_sc[row, :] = block_min
        arg_sc[row, :] = block_arg
        idx_ref[...] = block_arg

    @pl.when(k > 0)
    def _():
        upd = block_min < min_sc[row, :]
        new_min = jnp.where(upd, block_min, min_sc[row, :])
        new_arg = jnp.where(upd, block_arg, arg_sc[row, :])
        min_sc[row, :] = new_min
        arg_sc[row, :] = new_arg
        idx_ref[...] = new_arg


def _nearest_idx(z, embeddings):
    b, d = z.shape
    kk, _ = embeddings.shape
    kt, rb = _KT, _RB
    assert kk % kt == 0 and b % rb == 0
    return pl.pallas_call(
        functools.partial(_argmin_body, kt, rb),
        grid=(kk // kt, b // rb),
        in_specs=[
            pl.BlockSpec((rb, d), lambda k, s: (s, 0)),
            pl.BlockSpec((kt, d), lambda k, s: (k, 0)),
        ],
        out_specs=pl.BlockSpec((rb, 1), lambda k, s: (s, 0)),
        out_shape=jax.ShapeDtypeStruct((b, 1), jnp.int32),
        scratch_shapes=[
            pltpu.VMEM((b, 1), jnp.float32),
            pltpu.VMEM((b, 1), jnp.int32),
        ],
        compiler_params=pltpu.CompilerParams(
            dimension_semantics=("arbitrary", "arbitrary"),
        ),
    )(z, embeddings)


def _sc_gather(table, idx):
    kk, d = table.shape
    b = idx.shape[0]
    info = plsc.get_sparse_core_info()
    nw = info.num_cores * info.num_subcores
    assert b % (8 * nw) == 0
    b_per_w = b // nw
    mesh = plsc.VectorSubcoreMesh(core_axis_name="c", subcore_axis_name="s")

    @functools.partial(
        pl.kernel,
        mesh=mesh,
        out_type=jax.ShapeDtypeStruct((b, d), jnp.float32),
        scratch_types=[
            pltpu.VMEM((b_per_w,), jnp.int32),
            pltpu.VMEM((b_per_w, d), jnp.float32),
            pltpu.SemaphoreType.DMA,
        ],
    )
    def gather_kernel(table_hbm, idx_hbm, out_hbm, idx_v, rows_v, sem):
        wid = lax.axis_index("s") * info.num_cores + lax.axis_index("c")
        base = wid * b_per_w
        pltpu.sync_copy(idx_hbm.at[pl.ds(base, b_per_w)], idx_v)
        pltpu.async_copy(table_hbm.at[idx_v], rows_v, sem).wait()
        pltpu.sync_copy(rows_v, out_hbm.at[pl.ds(base, b_per_w)])

    return gather_kernel(table, idx)


def kernel(z, embeddings, batch_size):
    idx = _nearest_idx(z, embeddings).reshape(-1)
    return _sc_gather(embeddings, idx)


# speedup vs baseline: 1.3963x; 1.3963x over previous
import functools

import jax
import jax.numpy as jnp
from jax import lax
from jax.experimental import pallas as pl
from jax.experimental.pallas import tpu as pltpu
from jax.experimental.pallas import tpu_sc as plsc

_KT = 512
_RB = 256


def _argmin_body(kt, rb, zb_ref, eb_ref, ef_ref, idx_ref, min_sc, arg_sc, eh_sc):
    k = pl.program_id(0)
    s = pl.program_id(1)
    nk = pl.num_programs(0)
    nchunk = kt // 128

    @pl.when(s == 0)
    def _():
        ef = ef_ref[...]
        e_sq = ef * ef
        d = ef.shape[1]
        eh_sc[...] = 0.5 * lax.dot_general(
            jnp.ones((8, d), jnp.float32), e_sq, (((1,), (1,)), ((), ())),
            preferred_element_type=jnp.float32,
            precision=lax.Precision.HIGHEST,
        )[0:1, :]

    e_half = eh_sc[...]
    cross = lax.dot_general(
        zb_ref[...], eb_ref[...], (((1,), (1,)), ((), ())),
        preferred_element_type=jnp.float32,
    )

    row = pl.ds(s * rb, rb)
    rm = jnp.where(k == 0, 3.0e38, min_sc[row, :])
    ri = arg_sc[row, :]
    for c in range(nchunk):
        col0 = c * 128
        sc = e_half[:, col0:col0 + 128] - cross[:, col0:col0 + 128]
        m = sc < rm
        rm = jnp.where(m, sc, rm)
        ri = jnp.where(m, jnp.int32(k * nchunk + c), ri)
    min_sc[row, :] = rm
    arg_sc[row, :] = ri

    @pl.when(k == nk - 1)
    def _():
        lane = lax.broadcasted_iota(jnp.int32, (rb, 128), 1)
        gidx = ri * 128 + lane
        row_min = jnp.min(rm, axis=1, keepdims=True)
        cand = jnp.where(rm == row_min, gidx, jnp.int32(2**30))
        idx_ref[...] = jnp.min(cand, axis=1, keepdims=True)


def _nearest_idx(z, embeddings):
    b, d = z.shape
    kk, _ = embeddings.shape
    kt, rb = _KT, _RB
    assert kk % kt == 0 and b % rb == 0
    return pl.pallas_call(
        functools.partial(_argmin_body, kt, rb),
        grid=(kk // kt, b // rb),
        in_specs=[
            pl.BlockSpec((rb, d), lambda k, s: (s, 0)),
            pl.BlockSpec((kt, d), lambda k, s: (k, 0)),
            pl.BlockSpec((kt, d), lambda k, s: (k, 0)),
        ],
        out_specs=pl.BlockSpec((rb, 1), lambda k, s: (s, 0)),
        out_shape=jax.ShapeDtypeStruct((b, 1), jnp.int32),
        scratch_shapes=[
            pltpu.VMEM((b, 128), jnp.float32),
            pltpu.VMEM((b, 128), jnp.int32),
            pltpu.VMEM((1, kt), jnp.float32),
        ],
        compiler_params=pltpu.CompilerParams(
            dimension_semantics=("arbitrary", "arbitrary"),
        ),
    )(z.astype(jnp.bfloat16), embeddings.astype(jnp.bfloat16), embeddings)


def _sc_gather(table, idx):
    kk, d = table.shape
    b = idx.shape[0]
    info = plsc.get_sparse_core_info()
    nw = info.num_cores * info.num_subcores
    assert b % (8 * nw) == 0
    b_per_w = b // nw
    mesh = plsc.VectorSubcoreMesh(core_axis_name="c", subcore_axis_name="s")

    @functools.partial(
        pl.kernel,
        mesh=mesh,
        out_type=jax.ShapeDtypeStruct((b, d), jnp.float32),
        scratch_types=[
            pltpu.VMEM((b_per_w,), jnp.int32),
            pltpu.VMEM((b_per_w, d), jnp.float32),
            pltpu.SemaphoreType.DMA,
        ],
    )
    def gather_kernel(table_hbm, idx_hbm, out_hbm, idx_v, rows_v, sem):
        wid = lax.axis_index("s") * info.num_cores + lax.axis_index("c")
        base = wid * b_per_w
        pltpu.sync_copy(idx_hbm.at[pl.ds(base, b_per_w)], idx_v)
        pltpu.async_copy(table_hbm.at[idx_v], rows_v, sem).wait()
        pltpu.sync_copy(rows_v, out_hbm.at[pl.ds(base, b_per_w)])

    return gather_kernel(table, idx)


def kernel(z, embeddings, batch_size):
    idx = _nearest_idx(z, embeddings).reshape(-1)
    return _sc_gather(embeddings, idx)
